# baseline (speedup 1.0000x reference)
import functools

import jax
import jax.numpy as jnp
from jax import lax
from jax.experimental import pallas as pl
from jax.experimental.pallas import tpu as pltpu

MODEL_DIM = 2048
NUM_EXPERTS = 16
NUM_TOKENS = 8192
BLOCK_T = 512


def _gate_body(x_ref, w_ref, idx_ref, loc_ref, gate_ref, laux_ref,
               cnt_ref, me_ref):
    pid = pl.program_id(0)
    nblk = pl.num_programs(0)

    @pl.when(pid == 0)
    def _init():
        cnt_ref[...] = jnp.zeros_like(cnt_ref)
        me_ref[...] = jnp.zeros_like(me_ref)

    x = x_ref[...]
    w = w_ref[...]
    lg = lax.dot_general(w, x, (((1,), (1,)), ((), ())),
                         preferred_element_type=jnp.float32)

    m = jnp.max(lg, axis=0, keepdims=True)
    p = jnp.exp(lg - m)
    s = jnp.sum(p, axis=0, keepdims=True)
    inv_s = 1.0 / s
    gate_ref[...] = inv_s[0]

    si = lax.broadcasted_iota(jnp.int32, lg.shape, 0)
    eq = lg == m
    idx = jnp.min(jnp.where(eq, si, NUM_EXPERTS), axis=0)
    idx_ref[...] = idx.astype(jnp.int32)

    mask = (si == idx[None, :]).astype(jnp.float32)

    me_ref[...] = me_ref[...] + jnp.sum(p * inv_s, axis=1, keepdims=True)
    blk_cnt = jnp.sum(mask, axis=1, keepdims=True)

    cum = mask
    k = 1
    while k < BLOCK_T:
        z = jnp.zeros((NUM_EXPERTS, k), jnp.float32)
        cum = cum + jnp.concatenate([z, cum[:, :-k]], axis=1)
        k *= 2
    cum = cum - mask
    loc_in = jnp.sum(cum * mask, axis=0)
    offset = jnp.sum(cnt_ref[...] * mask, axis=0)
    loc_ref[...] = (loc_in + offset).astype(jnp.int32)

    cnt_ref[...] = cnt_ref[...] + blk_cnt

    @pl.when(pid == nblk - 1)
    def _fin():
        prod = me_ref[...] * cnt_ref[...]
        laux_ref[...] = jnp.sum(prod, axis=0, keepdims=True) * (
            NUM_EXPERTS / (NUM_TOKENS * NUM_TOKENS))


@jax.jit
def _top1_gate(x, W):
    nblk = NUM_TOKENS // BLOCK_T
    out_shapes = (
        jax.ShapeDtypeStruct((NUM_TOKENS,), jnp.int32),
        jax.ShapeDtypeStruct((NUM_TOKENS,), jnp.int32),
        jax.ShapeDtypeStruct((NUM_TOKENS,), jnp.float32),
        jax.ShapeDtypeStruct((1, 1), jnp.float32),
    )
    out = pl.pallas_call(
        _gate_body,
        grid=(nblk,),
        in_specs=[
            pl.BlockSpec((BLOCK_T, MODEL_DIM), lambda i: (i, 0)),
            pl.BlockSpec((NUM_EXPERTS, MODEL_DIM), lambda i: (0, 0)),
        ],
        out_specs=(
            pl.BlockSpec((BLOCK_T,), lambda i: (i,)),
            pl.BlockSpec((BLOCK_T,), lambda i: (i,)),
            pl.BlockSpec((BLOCK_T,), lambda i: (i,)),
            pl.BlockSpec((1, 1), lambda i: (0, 0)),
        ),
        out_shape=out_shapes,
        scratch_shapes=[
            pltpu.VMEM((NUM_EXPERTS, 1), jnp.float32),
            pltpu.VMEM((NUM_EXPERTS, 1), jnp.float32),
        ],
        compiler_params=pltpu.CompilerParams(
            dimension_semantics=("arbitrary",),
        ),
    )(x, W)
    idx, loc, gates1, laux = out
    return laux[0, 0], idx, loc, gates1


def kernel(x, W):
    laux, idx, loc, gates1 = _top1_gate(x, W)
    capacity = (NUM_TOKENS + NUM_EXPERTS - 1) // NUM_EXPERTS
    return (laux, idx, capacity, loc, gates1, NUM_EXPERTS)


# speedup vs baseline: 1.2123x; 1.2123x over previous
import functools

import jax
import jax.numpy as jnp
from jax import lax
from jax.experimental import pallas as pl
from jax.experimental.pallas import tpu as pltpu

MODEL_DIM = 2048
NUM_EXPERTS = 16
NUM_TOKENS = 8192
BLOCK_T = 1024


def _gate_body(x1_ref, x2_ref, w_ref, idx_ref, loc_ref, gate_ref, laux_ref,
               cnt_ref, me_ref):
    pid = pl.program_id(0)
    nblk = pl.num_programs(0)

    @pl.when(pid == 0)
    def _init():
        cnt_ref[...] = jnp.zeros_like(cnt_ref)
        me_ref[...] = jnp.zeros_like(me_ref)

    w = w_ref[...]
    lg1 = lax.dot_general(w, x1_ref[...], (((1,), (1,)), ((), ())),
                          preferred_element_type=jnp.float32)
    lg2 = lax.dot_general(w, x2_ref[...], (((1,), (1,)), ((), ())),
                          preferred_element_type=jnp.float32)
    lg = jnp.concatenate([lg1, lg2], axis=1)

    m = jnp.max(lg, axis=0, keepdims=True)
    p = jnp.exp(lg - m)
    s = jnp.sum(p, axis=0, keepdims=True)
    inv_s = 1.0 / s
    gate_ref[...] = inv_s[0]

    si = lax.broadcasted_iota(jnp.int32, lg.shape, 0)
    eq = lg == m
    idx = jnp.min(jnp.where(eq, si, NUM_EXPERTS), axis=0)
    idx_ref[...] = idx.astype(jnp.int32)

    mask = (si == idx[None, :]).astype(jnp.float32)

    me_ref[...] = me_ref[...] + jnp.sum(p * inv_s, axis=1, keepdims=True)
    blk_cnt = jnp.sum(mask, axis=1, keepdims=True)

    cum = mask
    k = 1
    while k < BLOCK_T:
        z = jnp.zeros((NUM_EXPERTS, k), jnp.float32)
        cum = cum + jnp.concatenate([z, cum[:, :-k]], axis=1)
        k *= 2
    cum = cum - mask
    loc_in = jnp.sum(cum * mask, axis=0)
    offset = jnp.sum(cnt_ref[...] * mask, axis=0)
    loc_ref[...] = (loc_in + offset).astype(jnp.int32)

    cnt_ref[...] = cnt_ref[...] + blk_cnt

    @pl.when(pid == nblk - 1)
    def _fin():
        prod = me_ref[...] * cnt_ref[...]
        laux_ref[...] = jnp.sum(prod, axis=0, keepdims=True) * (
            NUM_EXPERTS / (NUM_TOKENS * NUM_TOKENS))


@jax.jit
def _top1_gate(x, W):
    nblk = NUM_TOKENS // BLOCK_T
    out_shapes = (
        jax.ShapeDtypeStruct((NUM_TOKENS,), jnp.int32),
        jax.ShapeDtypeStruct((NUM_TOKENS,), jnp.int32),
        jax.ShapeDtypeStruct((NUM_TOKENS,), jnp.float32),
        jax.ShapeDtypeStruct((1, 1), jnp.float32),
    )
    out = pl.pallas_call(
        _gate_body,
        grid=(nblk,),
        in_specs=[
            pl.BlockSpec((BLOCK_T // 2, MODEL_DIM), lambda i: (2 * i, 0)),
            pl.BlockSpec((BLOCK_T // 2, MODEL_DIM), lambda i: (2 * i + 1, 0)),
            pl.BlockSpec((NUM_EXPERTS, MODEL_DIM), lambda i: (0, 0)),
        ],
        out_specs=(
            pl.BlockSpec((BLOCK_T,), lambda i: (i,)),
            pl.BlockSpec((BLOCK_T,), lambda i: (i,)),
            pl.BlockSpec((BLOCK_T,), lambda i: (i,)),
            pl.BlockSpec((1, 1), lambda i: (0, 0)),
        ),
        out_shape=out_shapes,
        scratch_shapes=[
            pltpu.VMEM((NUM_EXPERTS, 1), jnp.float32),
            pltpu.VMEM((NUM_EXPERTS, 1), jnp.float32),
        ],
        compiler_params=pltpu.CompilerParams(
            dimension_semantics=("arbitrary",),
        ),
    )(x, x, W)
    idx, loc, gates1, laux = out
    return laux[0, 0], idx, loc, gates1


def kernel(x, W):
    laux, idx, loc, gates1 = _top1_gate(x, W)
    capacity = (NUM_TOKENS + NUM_EXPERTS - 1) // NUM_EXPERTS
    return (laux, idx, capacity, loc, gates1, NUM_EXPERTS)
